# baseline (speedup 1.0000x reference)
import functools

import jax
import jax.numpy as jnp
from jax import lax
from jax.experimental import pallas as pl
from jax.experimental.pallas import tpu as pltpu
from jax.experimental.pallas import tpu_sc as plsc

_ROWS = 1_000_000
_DIM = 64
_WROWS = 500_000
_WDIM = 128
_NC = 2
_NS = 16
_NW = _NC * _NS
_CHUNK = 500
_NCHUNKS = _WROWS // _CHUNK
_MAX_PER_W = -(-_NCHUNKS // _NW)

_mesh = plsc.VectorSubcoreMesh(core_axis_name="c", subcore_axis_name="s")


@functools.partial(
    pl.kernel,
    out_type=jax.ShapeDtypeStruct((_WROWS, _WDIM), jnp.float32),
    mesh=_mesh,
    compiler_params=pltpu.CompilerParams(use_tc_tiling_on_sc=False),
    scratch_types=[
        pltpu.VMEM((2, _CHUNK, _WDIM), jnp.float32),
        pltpu.SemaphoreType.DMA,
        pltpu.SemaphoreType.DMA,
        pltpu.SemaphoreType.DMA,
        pltpu.SemaphoreType.DMA,
    ],
)
def _sc_copy(in_hbm, out_hbm, buf, in_sem0, in_sem1, out_sem0, out_sem1):
    wid = lax.axis_index("s") * _NC + lax.axis_index("c")
    in_sems = (in_sem0, in_sem1)
    out_sems = (out_sem0, out_sem1)

    def in_copy(g, b):
        base = (wid + g * _NW) * _CHUNK
        return pltpu.make_async_copy(
            in_hbm.at[pl.ds(base, _CHUNK), :], buf.at[b], in_sems[b]
        )

    def out_copy(g, b):
        base = (wid + g * _NW) * _CHUNK
        return pltpu.make_async_copy(
            buf.at[b], out_hbm.at[pl.ds(base, _CHUNK), :], out_sems[b]
        )

    def exists(g):
        return wid + g * _NW < _NCHUNKS

    pl.when(exists(0))(lambda: in_copy(0, 0).start())
    for g in range(_MAX_PER_W):
        b = g % 2
        if g >= 1:
            pl.when(exists(g - 1))(lambda g=g, b=b: out_copy(g - 1, 1 - b).wait())
        if g + 1 < _MAX_PER_W:
            pl.when(exists(g + 1))(lambda g=g, b=b: in_copy(g + 1, 1 - b).start())

        @pl.when(exists(g))
        def _(g=g, b=b):
            in_copy(g, b).wait()
            out_copy(g, b).start()

    g_last = _MAX_PER_W - 1
    pl.when(exists(g_last))(lambda: out_copy(g_last, g_last % 2).wait())


def kernel(path_emb):
    wide = jnp.reshape(path_emb, (_WROWS, _WDIM))
    out = _sc_copy(wide)
    return jnp.reshape(out, (_ROWS, _DIM))


# speedup vs baseline: 1.3574x; 1.3574x over previous
import jax
import jax.numpy as jnp
from jax.experimental import pallas as pl
from jax.experimental.pallas import tpu as pltpu

_ROWS = 1_000_000
_DIM = 64
_NQ = 8
_CHUNK = 5_000
_SUPER = _NQ * _CHUNK
_NSTEPS = _ROWS // _SUPER


def _copy_body(in_ref, out_ref, buf, *sems):
    in_sems = sems[:_NQ]
    out_sems = sems[_NQ:]
    for s in range(_NSTEPS):
        base = s * _SUPER
        ins = [
            pltpu.make_async_copy(
                in_ref.at[pl.ds(base + q * _CHUNK, _CHUNK), :],
                buf.at[q],
                in_sems[q],
            )
            for q in range(_NQ)
        ]
        outs = [
            pltpu.make_async_copy(
                buf.at[q],
                out_ref.at[pl.ds(base + q * _CHUNK, _CHUNK), :],
                out_sems[q],
            )
            for q in range(_NQ)
        ]
        for c in ins:
            c.start()
        for c in ins:
            c.wait()
        for c in outs:
            c.start()
        for c in outs:
            c.wait()


def kernel(path_emb):
    return pl.pallas_call(
        _copy_body,
        in_specs=[pl.BlockSpec(memory_space=pl.ANY)],
        out_specs=pl.BlockSpec(memory_space=pl.ANY),
        out_shape=jax.ShapeDtypeStruct((_ROWS, _DIM), jnp.float32),
        scratch_shapes=[pltpu.VMEM((_NQ, _CHUNK, _DIM), jnp.float32)]
        + [pltpu.SemaphoreType.DMA] * (2 * _NQ),
        compiler_params=pltpu.CompilerParams(
            vmem_limit_bytes=100 * 1024 * 1024,
        ),
    )(path_emb)
